# baseline (speedup 1.0000x reference)
import functools
import math

import jax
import jax.numpy as jnp
from jax.experimental import pallas as pl
from jax.experimental.pallas import tpu as pltpu
from jax.experimental.pallas import tpu_sc as plsc

H = 1024
NH = 16
NKV = 4
HD = 64
E = 8
K = 2
I = 1024
EPS = 1e-06
THETA = 1000000.0
S = 2048

TS = 256
NT = S // TS
NB = 23
NBTS = NB * TS
TRASH = S * K
NW = 32
RPW = NBTS // NW
LN_THETA = math.log(THETA)


def _rot_mat(n):
    i = jax.lax.broadcasted_iota(jnp.int32, (n, n), 0)
    c = jax.lax.broadcasted_iota(jnp.int32, (n, n), 1)
    cm = jnp.remainder(c, HD)
    neg = jnp.logical_and(i == c + HD // 2, cm < HD // 2)
    pos = jnp.logical_and(i == c - HD // 2, cm >= HD // 2)
    return jnp.where(neg, -1.0, 0.0) + jnp.where(pos, 1.0, 0.0)


def _cos_sin(t, n):
    r = jax.lax.broadcasted_iota(jnp.int32, (TS, n), 0).astype(jnp.float32)
    c = jax.lax.broadcasted_iota(jnp.int32, (TS, n), 1)
    fi = jnp.remainder(c, HD // 2).astype(jnp.float32)
    invf = jnp.exp(fi * (-LN_THETA / (HD // 2)))
    ang = (r + t * TS) * invf
    return jnp.cos(ang), jnp.sin(ang)


def _rmsnorm(x, w):
    v = jnp.mean(x * x, axis=-1, keepdims=True)
    return x * jax.lax.rsqrt(v + EPS) * w


def _dotT(a, b):
    return jax.lax.dot_general(a, b, (((1,), (1,)), ((), ())),
                               preferred_element_type=jnp.float32)


def _pre_kernel(x_ref, qw_ref, kw_ref, vw_ref, ln1_ref, q_ref, k_ref, v_ref):
    t = pl.program_id(0)
    xn = _rmsnorm(x_ref[...], ln1_ref[...])
    q = _dotT(xn, qw_ref[...])
    k = _dotT(xn, kw_ref[...])
    v = _dotT(xn, vw_ref[...])
    cos_q, sin_q = _cos_sin(t, NH * HD)
    rq = jnp.dot(q, _rot_mat(NH * HD), preferred_element_type=jnp.float32)
    q_ref[...] = q * cos_q + rq * sin_q
    cos_k, sin_k = _cos_sin(t, NKV * HD)
    rk = jnp.dot(k, _rot_mat(NKV * HD), preferred_element_type=jnp.float32)
    k_ref[...] = k * cos_k + rk * sin_k
    v_ref[...] = v


def _attn_kernel(q_ref, k_ref, v_ref, o_ref):
    q = q_ref[...]
    k = k_ref[...]
    v = v_ref[...]
    for h in range(NH):
        qh = q[:, h * HD:(h + 1) * HD]
        kv = h // (NH // NKV)
        kh = k[:, kv * HD:(kv + 1) * HD]
        vh = v[:, kv * HD:(kv + 1) * HD]
        s = _dotT(qh, kh) * (1.0 / math.sqrt(HD))
        m = jnp.max(s, axis=-1, keepdims=True)
        p = jnp.exp(s - m)
        p = p / jnp.sum(p, axis=-1, keepdims=True)
        o_ref[:, h * HD:(h + 1) * HD] = jnp.dot(
            p, vh, preferred_element_type=jnp.float32)


def _post_kernel(attn_ref, x_ref, ow_ref, ln2_ref, gw_ref,
                 h2_ref, x2_ref, rw_ref, sel_ref):
    o = _dotT(attn_ref[...], ow_ref[...])
    h2 = x_ref[...] + o
    h2_ref[...] = h2
    x2 = _rmsnorm(h2, ln2_ref[...])
    x2_ref[...] = x2
    logits = _dotT(x2, gw_ref[...])
    lm = jnp.max(logits, axis=-1, keepdims=True)
    el = jnp.exp(logits - lm)
    probs = el / jnp.sum(el, axis=-1, keepdims=True)
    iota = jax.lax.broadcasted_iota(jnp.int32, (TS, E), 1)
    m1 = jnp.max(probs, axis=-1, keepdims=True)
    i1 = jnp.min(jnp.where(probs == m1, iota, E), axis=-1, keepdims=True)
    probs2 = jnp.where(iota == i1, -jnp.inf, probs)
    m2 = jnp.max(probs2, axis=-1, keepdims=True)
    i2 = jnp.min(jnp.where(probs2 == m2, iota, E), axis=-1, keepdims=True)
    rw_ref[...] = jnp.concatenate([m1, m2], axis=1)
    sel_ref[...] = jnp.concatenate([i1, i2], axis=1)


def _group_kernel(be_ref, xs_ref, eg_ref, eu_ref, ed_ref, ys_ref):
    del be_ref
    x = xs_ref[...]
    g = _dotT(x, eg_ref[0])
    u = _dotT(x, eu_ref[0])
    hdn = g * jax.nn.sigmoid(g) * u
    ys_ref[...] = _dotT(hdn, ed_ref[0])


def _shared_kernel(x2_ref, sg_ref, su_ref, sd_ref, sgate_ref, sh_ref):
    x2 = x2_ref[...]
    g = _dotT(x2, sg_ref[...])
    u = _dotT(x2, su_ref[...])
    shared = _dotT(g * jax.nn.sigmoid(g) * u, sd_ref[...])
    gate = jax.nn.sigmoid(_dotT(x2, sgate_ref[...]))
    sh_ref[...] = gate * shared


def _final_kernel(h2_ref, y0_ref, y1_ref, rw_ref, sh_ref, out_ref):
    rw = rw_ref[...]
    moe = rw[:, 0:1] * y0_ref[...] + rw[:, 1:2] * y1_ref[...]
    out_ref[...] = h2_ref[...] + moe + sh_ref[...]


def _sc_dispatch(x2_hbm, idx_hbm, out_hbm, idx0, idx1, rows, sem):
    wid = jax.lax.axis_index("s") * 2 + jax.lax.axis_index("c")
    base = wid * RPW
    pltpu.sync_copy(idx_hbm.at[pl.ds(base, 96)], idx0)
    pltpu.async_copy(x2_hbm.at[idx0], rows, sem).wait()
    pltpu.sync_copy(rows, out_hbm.at[pl.ds(base, 96)])
    pltpu.sync_copy(idx_hbm.at[pl.ds(base + 96, 88)], idx1)
    pltpu.async_copy(x2_hbm.at[idx1], rows.at[pl.ds(0, 88)], sem).wait()
    pltpu.sync_copy(rows.at[pl.ds(0, 88)], out_hbm.at[pl.ds(base + 96, 88)])


def _sc_scatter(ys_hbm, dest_hbm, out_hbm, idx0, idx1, rows, sem):
    wid = jax.lax.axis_index("s") * 2 + jax.lax.axis_index("c")
    base = wid * RPW
    pltpu.sync_copy(dest_hbm.at[pl.ds(base, 96)], idx0)
    pltpu.sync_copy(ys_hbm.at[pl.ds(base, 96)], rows)
    pltpu.async_copy(rows, out_hbm.at[idx0], sem).wait()
    pltpu.sync_copy(dest_hbm.at[pl.ds(base + 96, 88)], idx1)
    pltpu.sync_copy(ys_hbm.at[pl.ds(base + 96, 88)], rows.at[pl.ds(0, 88)])
    pltpu.async_copy(rows.at[pl.ds(0, 88)], out_hbm.at[idx1], sem).wait()


def kernel(hidden_states, position_ids, q_w, k_w, v_w, o_w, ln1_w, ln2_w,
           gate_w, eg_w, eu_w, ed_w, sg_w, su_w, sd_w, sgate_w):
    x = hidden_states.reshape(S, H)

    q, k, v = pl.pallas_call(
        _pre_kernel,
        grid=(NT,),
        in_specs=[
            pl.BlockSpec((TS, H), lambda t: (t, 0)),
            pl.BlockSpec((NH * HD, H), lambda t: (0, 0)),
            pl.BlockSpec((NKV * HD, H), lambda t: (0, 0)),
            pl.BlockSpec((NKV * HD, H), lambda t: (0, 0)),
            pl.BlockSpec((H,), lambda t: (0,)),
        ],
        out_specs=[
            pl.BlockSpec((TS, NH * HD), lambda t: (t, 0)),
            pl.BlockSpec((TS, NKV * HD), lambda t: (t, 0)),
            pl.BlockSpec((TS, NKV * HD), lambda t: (t, 0)),
        ],
        out_shape=[
            jax.ShapeDtypeStruct((S, NH * HD), jnp.float32),
            jax.ShapeDtypeStruct((S, NKV * HD), jnp.float32),
            jax.ShapeDtypeStruct((S, NKV * HD), jnp.float32),
        ],
    )(x, q_w, k_w, v_w, ln1_w)

    attn = pl.pallas_call(
        _attn_kernel,
        grid=(NT,),
        in_specs=[
            pl.BlockSpec((TS, NH * HD), lambda t: (t, 0)),
            pl.BlockSpec((S, NKV * HD), lambda t: (0, 0)),
            pl.BlockSpec((S, NKV * HD), lambda t: (0, 0)),
        ],
        out_specs=pl.BlockSpec((TS, NH * HD), lambda t: (t, 0)),
        out_shape=jax.ShapeDtypeStruct((S, NH * HD), jnp.float32),
    )(q, k, v)

    h2, x2, rw, sel = pl.pallas_call(
        _post_kernel,
        grid=(NT,),
        in_specs=[
            pl.BlockSpec((TS, NH * HD), lambda t: (t, 0)),
            pl.BlockSpec((TS, H), lambda t: (t, 0)),
            pl.BlockSpec((H, NH * HD), lambda t: (0, 0)),
            pl.BlockSpec((H,), lambda t: (0,)),
            pl.BlockSpec((E, H), lambda t: (0, 0)),
        ],
        out_specs=[
            pl.BlockSpec((TS, H), lambda t: (t, 0)),
            pl.BlockSpec((TS, H), lambda t: (t, 0)),
            pl.BlockSpec((TS, K), lambda t: (t, 0)),
            pl.BlockSpec((TS, K), lambda t: (t, 0)),
        ],
        out_shape=[
            jax.ShapeDtypeStruct((S, H), jnp.float32),
            jax.ShapeDtypeStruct((S, H), jnp.float32),
            jax.ShapeDtypeStruct((S, K), jnp.float32),
            jax.ShapeDtypeStruct((S, K), jnp.int32),
        ],
    )(attn, x, o_w, ln2_w, gate_w)

    esel = sel.reshape(S * K)
    order = jnp.argsort(esel, stable=True)
    esorted = esel[order]
    counts = jnp.sum(esel[None, :] == jnp.arange(E, dtype=jnp.int32)[:, None],
                     axis=1)
    blocks_e = (counts + TS - 1) // TS
    cumblocks = jnp.cumsum(blocks_e)
    block_start = cumblocks - blocks_e
    block_expert = jnp.minimum(
        jnp.searchsorted(cumblocks, jnp.arange(NB), side="right"),
        E - 1).astype(jnp.int32)
    pad_start = (block_start * TS).astype(jnp.int32)
    first_idx = (jnp.cumsum(counts) - counts).astype(jnp.int32)
    slot = jnp.arange(S * K, dtype=jnp.int32)
    pos = pad_start[esorted] + (slot - first_idx[esorted])
    src_token = jnp.zeros((NBTS,), jnp.int32).at[pos].set(
        (order // K).astype(jnp.int32))
    dest = jnp.full((NBTS,), TRASH, jnp.int32).at[pos].set(
        ((order % K) * S + order // K).astype(jnp.int32))

    mesh = plsc.VectorSubcoreMesh(core_axis_name="c", subcore_axis_name="s")
    sc_scratch = [
        pltpu.VMEM((96,), jnp.int32),
        pltpu.VMEM((88,), jnp.int32),
        pltpu.VMEM((96, H), jnp.float32),
        pltpu.SemaphoreType.DMA,
    ]

    xs = pl.kernel(
        _sc_dispatch, mesh=mesh,
        out_type=jax.ShapeDtypeStruct((NBTS, H), jnp.float32),
        scratch_types=sc_scratch,
    )(x2, src_token)

    grid_spec = pltpu.PrefetchScalarGridSpec(
        num_scalar_prefetch=1,
        grid=(NB,),
        in_specs=[
            pl.BlockSpec((TS, H), lambda b, be: (b, 0)),
            pl.BlockSpec((1, I, H), lambda b, be: (be[b], 0, 0)),
            pl.BlockSpec((1, I, H), lambda b, be: (be[b], 0, 0)),
            pl.BlockSpec((1, H, I), lambda b, be: (be[b], 0, 0)),
        ],
        out_specs=pl.BlockSpec((TS, H), lambda b, be: (b, 0)),
    )
    ys = pl.pallas_call(
        _group_kernel,
        grid_spec=grid_spec,
        out_shape=jax.ShapeDtypeStruct((NBTS, H), jnp.float32),
    )(block_expert, xs, eg_w, eu_w, ed_w)

    ysu = pl.kernel(
        _sc_scatter, mesh=mesh,
        out_type=jax.ShapeDtypeStruct((S * K + 8, H), jnp.float32),
        scratch_types=sc_scratch,
    )(ys, dest)

    sh = pl.pallas_call(
        _shared_kernel,
        grid=(NT,),
        in_specs=[
            pl.BlockSpec((TS, H), lambda t: (t, 0)),
            pl.BlockSpec((I, H), lambda t: (0, 0)),
            pl.BlockSpec((I, H), lambda t: (0, 0)),
            pl.BlockSpec((H, I), lambda t: (0, 0)),
            pl.BlockSpec((1, H), lambda t: (0, 0)),
        ],
        out_specs=pl.BlockSpec((TS, H), lambda t: (t, 0)),
        out_shape=jax.ShapeDtypeStruct((S, H), jnp.float32),
    )(x2, sg_w, su_w, sd_w, sgate_w)

    out = pl.pallas_call(
        _final_kernel,
        grid=(NT,),
        in_specs=[
            pl.BlockSpec((TS, H), lambda t: (t, 0)),
            pl.BlockSpec((TS, H), lambda t: (t, 0)),
            pl.BlockSpec((TS, H), lambda t: (t + NT, 0)),
            pl.BlockSpec((TS, K), lambda t: (t, 0)),
            pl.BlockSpec((TS, H), lambda t: (t, 0)),
        ],
        out_specs=pl.BlockSpec((TS, H), lambda t: (t, 0)),
        out_shape=jax.ShapeDtypeStruct((S, H), jnp.float32),
    )(h2, ysu, ysu, rw, sh)

    return out.reshape(1, S, H)
